# Initial kernel scaffold; baseline (speedup 1.0000x reference)
import jax
import jax.numpy as jnp
from jax.experimental import pallas as pl


def kernel(class_logits, bbox_pred, tgt_labels, tgt_boxes):
    raise NotImplementedError("write your pallas kernel here")


# speedup vs baseline: 21.9849x; 21.9849x over previous
import jax
import jax.numpy as jnp
from jax.experimental import pallas as pl
from jax.experimental.pallas import tpu as pltpu

_NUM_CLASSES = 365
_C1 = _NUM_CLASSES + 1
_B, _Q, _T = 64, 300, 25
_EOS = 0.1
_W_CE, _W_BBOX, _W_GIOU = 1.0, 5.0, 2.0
_INF = 1e18
_MP = 304
_NP = 32

_INTERPRET = False


def _prep_kernel(logits_ref, bbox_ref, lab_ref, tbt_ref,
                 cost_ref, nll_lab_ref, nll_eos_ref):
    z = logits_ref[0]
    bb = bbox_ref[0]
    lab = lab_ref[0]
    tbt = tbt_ref[0]

    m = jnp.max(z, axis=1, keepdims=True)
    e = jnp.exp(z - m)
    s = jnp.sum(e, axis=1, keepdims=True)
    logs = jnp.log(s)

    cls_iota = jax.lax.broadcasted_iota(jnp.int32, (_C1, _T), 0)
    onehot = (cls_iota == lab).astype(jnp.float32)
    lablogit = jax.lax.dot(z, onehot,
                           precision=jax.lax.Precision.HIGHEST,
                           preferred_element_type=jnp.float32)
    cost_class = -(jnp.exp(lablogit - m) / s)

    cost_bbox = jnp.zeros((_Q, _T), jnp.float32)
    for k in range(4):
        cost_bbox = cost_bbox + jnp.abs(bb[:, k:k + 1] - tbt[k:k + 1, :])

    sx1 = bb[:, 0:1] - 0.5 * bb[:, 2:3]
    sy1 = bb[:, 1:2] - 0.5 * bb[:, 3:4]
    sx2 = bb[:, 0:1] + 0.5 * bb[:, 2:3]
    sy2 = bb[:, 1:2] + 0.5 * bb[:, 3:4]
    tx1 = tbt[0:1, :] - 0.5 * tbt[2:3, :]
    ty1 = tbt[1:2, :] - 0.5 * tbt[3:4, :]
    tx2 = tbt[0:1, :] + 0.5 * tbt[2:3, :]
    ty2 = tbt[1:2, :] + 0.5 * tbt[3:4, :]
    area1 = (sx2 - sx1) * (sy2 - sy1)
    area2 = (tx2 - tx1) * (ty2 - ty1)
    ltx = jnp.maximum(sx1, tx1)
    lty = jnp.maximum(sy1, ty1)
    rbx = jnp.minimum(sx2, tx2)
    rby = jnp.minimum(sy2, ty2)
    inter = jnp.clip(rbx - ltx, 0.0) * jnp.clip(rby - lty, 0.0)
    union = area1 + area2 - inter
    iou = inter / union
    ltix = jnp.minimum(sx1, tx1)
    ltiy = jnp.minimum(sy1, ty1)
    rbix = jnp.maximum(sx2, tx2)
    rbiy = jnp.maximum(sy2, ty2)
    areai = jnp.clip(rbix - ltix, 0.0) * jnp.clip(rbiy - ltiy, 0.0)
    cost_giou = -(iou - (areai - union) / areai)

    cost = (_W_BBOX * cost_bbox + _W_CE * cost_class
            + _W_GIOU * cost_giou)

    cost_ref[...] = jnp.full((1, _NP, _MP), _INF, jnp.float32)
    cost_ref[0, 0:_T, 1:_Q + 1] = cost.T

    nll_lab_ref[0] = -(lablogit - m - logs).T
    zeos = z[:, _C1 - 1:_C1]
    nll_eos_ref[0] = -(zeos - m - logs).T


def _solve_kernel(cost_ref, nll_lab_ref, nll_eos_ref, bbt_ref, tbt_ref,
                  out_total, out_ce, out_bb, out_gi,
                  u_ref, v_ref, p_ref, way_ref, minv_ref, used_ref,
                  urow_ref, j0_ref):
    INF = jnp.float32(_INF)
    col = jax.lax.broadcasted_iota(jnp.int32, (_B, _MP), 1)
    rowi = jax.lax.broadcasted_iota(jnp.int32, (_B, _NP), 1)
    riota = jax.lax.broadcasted_iota(jnp.int32, (_B, _NP, 1), 1)

    def vsel_i(a, j):
        return jnp.sum(jnp.where(col == j, a, 0), axis=1, keepdims=True)

    u_ref[...] = jnp.zeros((_B, _NP), jnp.float32)
    v_ref[...] = jnp.zeros((_B, _MP), jnp.float32)
    p_ref[...] = jnp.zeros((_B, _MP), jnp.int32)
    way_ref[...] = jnp.zeros((_B, _MP), jnp.int32)

    def row_step(i, carry):
        p_ref[...] = jnp.where(col == 0, i, p_ref[...])
        minv_ref[...] = jnp.full((_B, _MP), INF, jnp.float32)
        used_ref[...] = jnp.zeros((_B, _MP), jnp.int32)
        urow_ref[...] = jnp.zeros((_B, _NP), jnp.int32)
        j0_ref[...] = jnp.zeros((_B, 1), jnp.int32)

        def body(cont):
            p = p_ref[...]
            j0 = j0_ref[...]
            i0 = vsel_i(p, j0)
            active = i0 != 0
            used = (used_ref[...] != 0) | ((col == j0) & active)
            used_ref[...] = used.astype(jnp.int32)
            urow = (urow_ref[...] != 0) | ((rowi == i0) & active)
            urow_ref[...] = urow.astype(jnp.int32)
            u = u_ref[...]
            v = v_ref[...]
            minv = minv_ref[...]
            way = way_ref[...]
            ui0 = jnp.sum(jnp.where(rowi == i0, u, 0.0), axis=1, keepdims=True)
            rsel = riota == (i0 - 1)[:, :, None]
            crow = jnp.sum(jnp.where(rsel, cost_ref[...], 0.0), axis=1)
            cur = crow - ui0 - v
            mask = (~used) & (cur < minv) & (col >= 1) & active
            minv = jnp.where(mask, cur, minv)
            way_ref[...] = jnp.where(mask, j0, way)
            cand = jnp.where(used, INF, minv)
            delta = jnp.min(cand, axis=1, keepdims=True)
            j1 = jnp.argmin(cand, axis=1).astype(jnp.int32)[:, None]
            u_ref[...] = u + jnp.where(urow & active, delta, 0.0)
            v_ref[...] = jnp.where(used & active, v - delta, v)
            minv_ref[...] = jnp.where((~used) & active, minv - delta, minv)
            j0 = jnp.where(active, j1, j0)
            j0_ref[...] = j0
            return jnp.any(vsel_i(p, j0) != 0)

        jax.lax.while_loop(lambda c: c, body, jnp.bool_(True))

        def bbody(cont):
            p = p_ref[...]
            way = way_ref[...]
            j0 = j0_ref[...]
            active = j0 != 0
            j1 = vsel_i(way, j0)
            pj1 = vsel_i(p, j1)
            p_ref[...] = jnp.where((col == j0) & active, pj1, p)
            j0 = jnp.where(active, j1, j0)
            j0_ref[...] = j0
            return jnp.any(j0 != 0)

        jax.lax.while_loop(lambda c: c, bbody, jnp.bool_(True))
        return carry

    jax.lax.fori_loop(1, _T + 1, row_step, jnp.int32(0))
    p = p_ref[...]

    pmat = p[:, 1:_Q + 1]
    tio = jax.lax.broadcasted_iota(jnp.int32, (_B, _T, _Q), 1)
    onehot = (pmat[:, None, :] == tio + 1).astype(jnp.float32)

    nll_lab = nll_lab_ref[...]
    nll_eos = nll_eos_ref[...]
    ce_matched = jnp.sum(onehot * nll_lab)
    eos_matched = jnp.sum(onehot * nll_eos)
    eos_all = jnp.sum(nll_eos)
    denom = _EOS * (_B * _Q - _B * _T) + 1.0 * (_B * _T)
    loss_ce = (_EOS * (eos_all - eos_matched) + ce_matched) / denom

    bbt = bbt_ref[...]
    tbt = tbt_ref[...]
    sb = [jnp.sum(onehot * bbt[k][:, None, :], axis=2) for k in range(4)]
    tb = [tbt[k] for k in range(4)]

    l1 = jnp.float32(0.0)
    for k in range(4):
        l1 = l1 + jnp.sum(jnp.abs(sb[k] - tb[k]))
    loss_bbox = l1 / _B

    sx1 = sb[0] - 0.5 * sb[2]
    sy1 = sb[1] - 0.5 * sb[3]
    sx2 = sb[0] + 0.5 * sb[2]
    sy2 = sb[1] + 0.5 * sb[3]
    tx1 = tb[0] - 0.5 * tb[2]
    ty1 = tb[1] - 0.5 * tb[3]
    tx2 = tb[0] + 0.5 * tb[2]
    ty2 = tb[1] + 0.5 * tb[3]
    area1 = (sx2 - sx1) * (sy2 - sy1)
    area2 = (tx2 - tx1) * (ty2 - ty1)
    inter = (jnp.clip(jnp.minimum(sx2, tx2) - jnp.maximum(sx1, tx1), 0.0)
             * jnp.clip(jnp.minimum(sy2, ty2) - jnp.maximum(sy1, ty1), 0.0))
    union = area1 + area2 - inter
    iou = inter / union
    areai = (jnp.clip(jnp.maximum(sx2, tx2) - jnp.minimum(sx1, tx1), 0.0)
             * jnp.clip(jnp.maximum(sy2, ty2) - jnp.minimum(sy1, ty1), 0.0))
    giou = iou - (areai - union) / areai
    loss_giou = jnp.sum(1.0 - giou) / _B

    total = _W_CE * loss_ce + _W_BBOX * loss_bbox + _W_GIOU * loss_giou
    out_total[...] = total.reshape(1, 1)
    out_ce[...] = loss_ce.reshape(1, 1)
    out_bb[...] = loss_bbox.reshape(1, 1)
    out_gi[...] = loss_giou.reshape(1, 1)


def kernel(class_logits, bbox_pred, tgt_labels, tgt_boxes):
    lab3 = tgt_labels.astype(jnp.int32).reshape(_B, 1, _T)
    tbt = jnp.transpose(tgt_boxes, (0, 2, 1))

    cost, nll_lab, nll_eos = pl.pallas_call(
        _prep_kernel,
        grid=(_B,),
        in_specs=[
            pl.BlockSpec((1, _Q, _C1), lambda b: (b, 0, 0)),
            pl.BlockSpec((1, _Q, 4), lambda b: (b, 0, 0)),
            pl.BlockSpec((1, 1, _T), lambda b: (b, 0, 0)),
            pl.BlockSpec((1, 4, _T), lambda b: (b, 0, 0)),
        ],
        out_specs=[
            pl.BlockSpec((1, _NP, _MP), lambda b: (b, 0, 0)),
            pl.BlockSpec((1, _T, _Q), lambda b: (b, 0, 0)),
            pl.BlockSpec((1, 1, _Q), lambda b: (b, 0, 0)),
        ],
        out_shape=[
            jax.ShapeDtypeStruct((_B, _NP, _MP), jnp.float32),
            jax.ShapeDtypeStruct((_B, _T, _Q), jnp.float32),
            jax.ShapeDtypeStruct((_B, 1, _Q), jnp.float32),
        ],
        interpret=_INTERPRET,
    )(class_logits, bbox_pred, lab3, tbt)

    bbt = jnp.transpose(bbox_pred, (2, 0, 1))
    tbt2 = jnp.transpose(tgt_boxes, (2, 0, 1))

    outs = pl.pallas_call(
        _solve_kernel,
        out_shape=[jax.ShapeDtypeStruct((1, 1), jnp.float32)] * 4,
        scratch_shapes=[
            pltpu.VMEM((_B, _NP), jnp.float32),
            pltpu.VMEM((_B, _MP), jnp.float32),
            pltpu.VMEM((_B, _MP), jnp.int32),
            pltpu.VMEM((_B, _MP), jnp.int32),
            pltpu.VMEM((_B, _MP), jnp.float32),
            pltpu.VMEM((_B, _MP), jnp.int32),
            pltpu.VMEM((_B, _NP), jnp.int32),
            pltpu.VMEM((_B, 1), jnp.int32),
        ],
        interpret=_INTERPRET,
    )(cost, nll_lab, nll_eos, bbt, tbt2)

    total, ce, bb, gi = [o[0, 0] for o in outs]
    return (total, ce, bb, gi)


# speedup vs baseline: 27.0018x; 1.2282x over previous
import functools

import jax
import jax.numpy as jnp
from jax import lax
from jax.experimental import pallas as pl
from jax.experimental.pallas import tpu as pltpu
from jax.experimental.pallas import tpu_sc as plsc

_NUM_CLASSES = 365
_C1 = _NUM_CLASSES + 1
_B, _Q, _T = 64, 300, 25
_EOS = 0.1
_W_CE, _W_BBOX, _W_GIOU = 1.0, 5.0, 2.0
_INF = 1e18
_MP = 304
_NP = 32
_NCHUNK = _MP // 16

_INTERPRET = False


def _prep_kernel(logits_ref, bbox_ref, lab_ref, tbt_ref,
                 cost_ref, nll_lab_ref, nll_eos_ref):
    z = logits_ref[0]
    bb = bbox_ref[0]
    lab = lab_ref[0]
    tbt = tbt_ref[0]

    m = jnp.max(z, axis=1, keepdims=True)
    e = jnp.exp(z - m)
    s = jnp.sum(e, axis=1, keepdims=True)
    logs = jnp.log(s)

    cls_iota = jax.lax.broadcasted_iota(jnp.int32, (_C1, _T), 0)
    onehot = (cls_iota == lab).astype(jnp.float32)
    lablogit = jax.lax.dot(z, onehot,
                           precision=jax.lax.Precision.HIGHEST,
                           preferred_element_type=jnp.float32)
    cost_class = -(jnp.exp(lablogit - m) / s)

    cost_bbox = jnp.zeros((_Q, _T), jnp.float32)
    for k in range(4):
        cost_bbox = cost_bbox + jnp.abs(bb[:, k:k + 1] - tbt[k:k + 1, :])

    sx1 = bb[:, 0:1] - 0.5 * bb[:, 2:3]
    sy1 = bb[:, 1:2] - 0.5 * bb[:, 3:4]
    sx2 = bb[:, 0:1] + 0.5 * bb[:, 2:3]
    sy2 = bb[:, 1:2] + 0.5 * bb[:, 3:4]
    tx1 = tbt[0:1, :] - 0.5 * tbt[2:3, :]
    ty1 = tbt[1:2, :] - 0.5 * tbt[3:4, :]
    tx2 = tbt[0:1, :] + 0.5 * tbt[2:3, :]
    ty2 = tbt[1:2, :] + 0.5 * tbt[3:4, :]
    area1 = (sx2 - sx1) * (sy2 - sy1)
    area2 = (tx2 - tx1) * (ty2 - ty1)
    ltx = jnp.maximum(sx1, tx1)
    lty = jnp.maximum(sy1, ty1)
    rbx = jnp.minimum(sx2, tx2)
    rby = jnp.minimum(sy2, ty2)
    inter = jnp.clip(rbx - ltx, 0.0) * jnp.clip(rby - lty, 0.0)
    union = area1 + area2 - inter
    iou = inter / union
    ltix = jnp.minimum(sx1, tx1)
    ltiy = jnp.minimum(sy1, ty1)
    rbix = jnp.maximum(sx2, tx2)
    rbiy = jnp.maximum(sy2, ty2)
    areai = jnp.clip(rbix - ltix, 0.0) * jnp.clip(rbiy - ltiy, 0.0)
    cost_giou = -(iou - (areai - union) / areai)

    cost = (_W_BBOX * cost_bbox + _W_CE * cost_class
            + _W_GIOU * cost_giou)

    cost_ref[...] = jnp.full((1, _NP, _MP), _INF, jnp.float32)
    cost_ref[0, 0:_T, 1:_Q + 1] = cost.T

    nll_lab_ref[0] = -(lablogit - m - logs).T
    zeos = z[:, _C1 - 1:_C1]
    nll_eos_ref[0] = -(zeos - m - logs).T


def _sc_sstore(ref, idx, val, dtype):
    lane0 = lax.iota(jnp.int32, 16) == 0
    iv = jnp.zeros((16,), jnp.int32) + idx
    xv = jnp.zeros((16,), dtype) + val
    plsc.store_scatter(ref, [iv], xv, mask=lane0)


def _sc_sload(ref, idx):
    iv = jnp.zeros((16,), jnp.int32) + idx
    return plsc.load_gather(ref, [iv])[0]


def _sc_hung_body(cost_hbm, out_hbm, cost_v, u_v, v_v, p_v, way_v,
                  minv_v, used_v, urow_v, ans_v, scal_s):
    INF = jnp.float32(_INF)
    wid = lax.axis_index("s") * 2 + lax.axis_index("c")

    for img in range(2):
        b = wid * 2 + img
        pltpu.sync_copy(cost_hbm.at[b], cost_v)

        for k in range(_NCHUNK):
            ds = pl.ds(k * 16, 16)
            v_v[ds] = jnp.zeros((16,), jnp.float32)
            p_v[ds] = jnp.zeros((16,), jnp.int32)
            way_v[ds] = jnp.zeros((16,), jnp.int32)
        for k in range(_NP // 16):
            u_v[pl.ds(k * 16, 16)] = jnp.zeros((16,), jnp.float32)

        def row_step(i, _):
            for k in range(_NCHUNK):
                ds = pl.ds(k * 16, 16)
                minv_v[ds] = jnp.full((16,), INF, jnp.float32)
                used_v[ds] = jnp.zeros((16,), jnp.int32)
            for k in range(_NP // 16):
                ds = pl.ds(k * 16, 16)
                io = lax.iota(jnp.int32, 16) + k * 16
                urow_v[ds] = jnp.where(io == i, 1, 0)

            scal_s[0] = jnp.int32(0)
            scal_s[1] = i

            def sbody(it, _):
                i0 = scal_s[1]

                @pl.when(i0 != 0)
                def _():
                    j0 = scal_s[0]
                    _sc_sstore(used_v, j0, 1, jnp.int32)
                    _sc_sstore(urow_v, i0, 1, jnp.int32)
                    ui0 = _sc_sload(u_v, i0)
                    delta = INF
                    j1 = jnp.int32(0)
                    for k in range(_NCHUNK):
                        ds = pl.ds(k * 16, 16)
                        cchunk = cost_v[i0 - 1, ds]
                        vcur = cchunk - ui0 - v_v[ds]
                        uch = used_v[ds]
                        mch = minv_v[ds]
                        mask = (uch == 0) & (vcur < mch)
                        mch = jnp.where(mask, vcur, mch)
                        minv_v[ds] = mch
                        way_v[ds] = jnp.where(mask, j0, way_v[ds])
                        cand = jnp.where(uch == 0, mch, INF)
                        cmin = jnp.min(cand)
                        cidx = jnp.min(plsc.all_reduce_ffs(cand == cmin))
                        better = cmin < delta
                        j1 = jnp.where(better, cidx + k * 16, j1)
                        delta = jnp.where(better, cmin, delta)
                    for k in range(_NP // 16):
                        ds = pl.ds(k * 16, 16)
                        u_v[ds] = u_v[ds] + jnp.where(
                            urow_v[ds] != 0, delta, jnp.float32(0.0))
                    for k in range(_NCHUNK):
                        ds = pl.ds(k * 16, 16)
                        um = used_v[ds] != 0
                        vv = v_v[ds]
                        mv = minv_v[ds]
                        v_v[ds] = jnp.where(um, vv - delta, vv)
                        minv_v[ds] = jnp.where(um, mv, mv - delta)
                    scal_s[0] = j1
                    scal_s[1] = _sc_sload(p_v, j1)
                return 0

            lax.fori_loop(0, i, sbody, 0)

            def bbody(it, _):
                j0c = scal_s[0]

                @pl.when(j0c != 0)
                def _():
                    j1w = _sc_sload(way_v, j0c)
                    pj = jnp.where(j1w == 0, i, _sc_sload(p_v, j1w))
                    _sc_sstore(p_v, j0c, pj, jnp.int32)
                    scal_s[0] = j1w
                return 0

            lax.fori_loop(0, i, bbody, 0)
            return 0

        lax.fori_loop(1, _T + 1, row_step, 0)

        for k in range(_NCHUNK):
            ds = pl.ds(k * 16, 16)
            pj = p_v[ds]
            idx = lax.iota(jnp.int32, 16) + k * 16
            mask = (pj > 0) & (idx >= 1) & (idx <= _Q)
            plsc.store_scatter(ans_v, [pj - 1], idx - 1, mask=mask)
        pltpu.sync_copy(ans_v, out_hbm.at[b])


def _sc_hungarian(cost):
    mesh = plsc.VectorSubcoreMesh(core_axis_name="c", subcore_axis_name="s")
    f = functools.partial(
        pl.kernel,
        mesh=mesh,
        compiler_params=pltpu.CompilerParams(needs_layout_passes=False),
        out_type=jax.ShapeDtypeStruct((_B, _NP), jnp.int32),
        scratch_types=[
            pltpu.VMEM((_NP, _MP), jnp.float32),
            pltpu.VMEM((_NP,), jnp.float32),
            pltpu.VMEM((_MP,), jnp.float32),
            pltpu.VMEM((_MP,), jnp.int32),
            pltpu.VMEM((_MP,), jnp.int32),
            pltpu.VMEM((_MP,), jnp.float32),
            pltpu.VMEM((_MP,), jnp.int32),
            pltpu.VMEM((_NP,), jnp.int32),
            pltpu.VMEM((_NP,), jnp.int32),
            pltpu.SMEM((2,), jnp.int32),
        ],
    )(_sc_hung_body)
    return f(cost)


def _loss_kernel(src_ref, nll_lab_ref, nll_eos_ref, bbt_ref, tbt_ref,
                 out_total, out_ce, out_bb, out_gi):
    colq = jax.lax.broadcasted_iota(jnp.int32, (_B, _Q), 1)
    tbt = tbt_ref[...]

    ce_q = jnp.zeros((_B, _Q), jnp.float32)
    matched = jnp.zeros((_B, _Q), jnp.bool_)
    tbq = [jnp.zeros((_B, _Q), jnp.float32) for _ in range(4)]
    for t in range(_T):
        sel = colq == src_ref[:, t:t + 1]
        matched = matched | sel
        ce_q = jnp.where(sel, nll_lab_ref[:, t, :], ce_q)
        for k in range(4):
            tbq[k] = jnp.where(sel, tbt[k][:, t:t + 1], tbq[k])

    nll_eos = nll_eos_ref[:, 0, :]
    ce_matched = jnp.sum(ce_q)
    eos_matched = jnp.sum(jnp.where(matched, nll_eos, 0.0))
    eos_all = jnp.sum(nll_eos)
    denom = _EOS * (_B * _Q - _B * _T) + 1.0 * (_B * _T)
    loss_ce = (_EOS * (eos_all - eos_matched) + ce_matched) / denom

    bb = [bbt_ref[k] for k in range(4)]
    l1 = jnp.zeros((_B, _Q), jnp.float32)
    for k in range(4):
        l1 = l1 + jnp.abs(bb[k] - tbq[k])
    loss_bbox = jnp.sum(jnp.where(matched, l1, 0.0)) / _B

    sx1 = bb[0] - 0.5 * bb[2]
    sy1 = bb[1] - 0.5 * bb[3]
    sx2 = bb[0] + 0.5 * bb[2]
    sy2 = bb[1] + 0.5 * bb[3]
    tx1 = tbq[0] - 0.5 * tbq[2]
    ty1 = tbq[1] - 0.5 * tbq[3]
    tx2 = tbq[0] + 0.5 * tbq[2]
    ty2 = tbq[1] + 0.5 * tbq[3]
    area1 = (sx2 - sx1) * (sy2 - sy1)
    area2 = (tx2 - tx1) * (ty2 - ty1)
    inter = (jnp.clip(jnp.minimum(sx2, tx2) - jnp.maximum(sx1, tx1), 0.0)
             * jnp.clip(jnp.minimum(sy2, ty2) - jnp.maximum(sy1, ty1), 0.0))
    union = area1 + area2 - inter
    iou = inter / union
    areai = (jnp.clip(jnp.maximum(sx2, tx2) - jnp.minimum(sx1, tx1), 0.0)
             * jnp.clip(jnp.maximum(sy2, ty2) - jnp.minimum(sy1, ty1), 0.0))
    giou = iou - (areai - union) / areai
    loss_giou = jnp.sum(jnp.where(matched, 1.0 - giou, 0.0)) / _B

    total = _W_CE * loss_ce + _W_BBOX * loss_bbox + _W_GIOU * loss_giou
    out_total[...] = total.reshape(1, 1)
    out_ce[...] = loss_ce.reshape(1, 1)
    out_bb[...] = loss_bbox.reshape(1, 1)
    out_gi[...] = loss_giou.reshape(1, 1)


def kernel(class_logits, bbox_pred, tgt_labels, tgt_boxes):
    lab3 = tgt_labels.astype(jnp.int32).reshape(_B, 1, _T)
    tbt = jnp.transpose(tgt_boxes, (0, 2, 1))

    cost, nll_lab, nll_eos = pl.pallas_call(
        _prep_kernel,
        grid=(_B,),
        in_specs=[
            pl.BlockSpec((1, _Q, _C1), lambda b: (b, 0, 0)),
            pl.BlockSpec((1, _Q, 4), lambda b: (b, 0, 0)),
            pl.BlockSpec((1, 1, _T), lambda b: (b, 0, 0)),
            pl.BlockSpec((1, 4, _T), lambda b: (b, 0, 0)),
        ],
        out_specs=[
            pl.BlockSpec((1, _NP, _MP), lambda b: (b, 0, 0)),
            pl.BlockSpec((1, _T, _Q), lambda b: (b, 0, 0)),
            pl.BlockSpec((1, 1, _Q), lambda b: (b, 0, 0)),
        ],
        out_shape=[
            jax.ShapeDtypeStruct((_B, _NP, _MP), jnp.float32),
            jax.ShapeDtypeStruct((_B, _T, _Q), jnp.float32),
            jax.ShapeDtypeStruct((_B, 1, _Q), jnp.float32),
        ],
        interpret=_INTERPRET,
    )(class_logits, bbox_pred, lab3, tbt)

    src = _sc_hungarian(cost)

    bbt = jnp.transpose(bbox_pred, (2, 0, 1))
    tbt2 = jnp.transpose(tgt_boxes, (2, 0, 1))

    outs = pl.pallas_call(
        _loss_kernel,
        out_shape=[jax.ShapeDtypeStruct((1, 1), jnp.float32)] * 4,
        interpret=_INTERPRET,
    )(src, nll_lab, nll_eos, bbt, tbt2)

    total, ce, bb, gi = [o[0, 0] for o in outs]
    return (total, ce, bb, gi)


# speedup vs baseline: 29.0827x; 1.0771x over previous
import functools

import jax
import jax.numpy as jnp
from jax import lax
from jax.experimental import pallas as pl
from jax.experimental.pallas import tpu as pltpu
from jax.experimental.pallas import tpu_sc as plsc

_NUM_CLASSES = 365
_C1 = _NUM_CLASSES + 1
_B, _Q, _T = 64, 300, 25
_EOS = 0.1
_W_CE, _W_BBOX, _W_GIOU = 1.0, 5.0, 2.0
_INF = 1e18
_MP = 304
_NP = 32
_NCHUNK = _MP // 16

_INTERPRET = False


_IMG = 4


def _prep_one(z, bb, lab, tbt, cost_ref, nll_lab_ref, nll_eos_ref, img):
    m = jnp.max(z, axis=1, keepdims=True)
    e = jnp.exp(z - m)
    s = jnp.sum(e, axis=1, keepdims=True)
    logs = jnp.log(s)

    cls_iota = jax.lax.broadcasted_iota(jnp.int32, (_C1, _T), 0)
    onehot = (cls_iota == lab).astype(jnp.bfloat16)
    h1 = z.astype(jnp.bfloat16)
    r1 = z - h1.astype(jnp.float32)
    h2 = r1.astype(jnp.bfloat16)
    h3 = (r1 - h2.astype(jnp.float32)).astype(jnp.bfloat16)
    lablogit = (jax.lax.dot(h1, onehot, preferred_element_type=jnp.float32)
                + jax.lax.dot(h2, onehot, preferred_element_type=jnp.float32)
                + jax.lax.dot(h3, onehot, preferred_element_type=jnp.float32))
    cost_class = -(jnp.exp(lablogit - m) / s)

    cost_bbox = jnp.zeros((_Q, _T), jnp.float32)
    for k in range(4):
        cost_bbox = cost_bbox + jnp.abs(bb[:, k:k + 1] - tbt[k:k + 1, :])

    sx1 = bb[:, 0:1] - 0.5 * bb[:, 2:3]
    sy1 = bb[:, 1:2] - 0.5 * bb[:, 3:4]
    sx2 = bb[:, 0:1] + 0.5 * bb[:, 2:3]
    sy2 = bb[:, 1:2] + 0.5 * bb[:, 3:4]
    tx1 = tbt[0:1, :] - 0.5 * tbt[2:3, :]
    ty1 = tbt[1:2, :] - 0.5 * tbt[3:4, :]
    tx2 = tbt[0:1, :] + 0.5 * tbt[2:3, :]
    ty2 = tbt[1:2, :] + 0.5 * tbt[3:4, :]
    area1 = (sx2 - sx1) * (sy2 - sy1)
    area2 = (tx2 - tx1) * (ty2 - ty1)
    ltx = jnp.maximum(sx1, tx1)
    lty = jnp.maximum(sy1, ty1)
    rbx = jnp.minimum(sx2, tx2)
    rby = jnp.minimum(sy2, ty2)
    inter = jnp.clip(rbx - ltx, 0.0) * jnp.clip(rby - lty, 0.0)
    union = area1 + area2 - inter
    iou = inter / union
    ltix = jnp.minimum(sx1, tx1)
    ltiy = jnp.minimum(sy1, ty1)
    rbix = jnp.maximum(sx2, tx2)
    rbiy = jnp.maximum(sy2, ty2)
    areai = jnp.clip(rbix - ltix, 0.0) * jnp.clip(rbiy - ltiy, 0.0)
    cost_giou = -(iou - (areai - union) / areai)

    cost = (_W_BBOX * cost_bbox + _W_CE * cost_class
            + _W_GIOU * cost_giou)

    cost_ref[img] = jnp.full((_NP, _MP), _INF, jnp.float32)
    cost_ref[img, 0:_T, 1:_Q + 1] = cost.T

    nll_lab_ref[img] = -(lablogit - m - logs).T
    zeos = z[:, _C1 - 1:_C1]
    nll_eos_ref[img] = -(zeos - m - logs).T


def _prep_kernel(logits_ref, bbox_ref, lab_ref, tbt_ref,
                 cost_ref, nll_lab_ref, nll_eos_ref):
    for img in range(_IMG):
        _prep_one(logits_ref[img], bbox_ref[img], lab_ref[img],
                  tbt_ref[img], cost_ref, nll_lab_ref, nll_eos_ref, img)


def _sc_sstore(ref, idx, val, dtype):
    lane0 = lax.iota(jnp.int32, 16) == 0
    iv = jnp.zeros((16,), jnp.int32) + idx
    xv = jnp.zeros((16,), dtype) + val
    plsc.store_scatter(ref, [iv], xv, mask=lane0)


def _sc_sload(ref, idx):
    iv = jnp.zeros((16,), jnp.int32) + idx
    return plsc.load_gather(ref, [iv])[0]


def _sc_hung_body(cost_hbm, out_hbm, cost_v, u_v, v_v, p_v, way_v,
                  minv_v, used_v, urow_v, ans_v, scal_s):
    INF = jnp.float32(_INF)
    wid = lax.axis_index("s") * 2 + lax.axis_index("c")

    for img in range(2):
        b = wid * 2 + img
        pltpu.sync_copy(cost_hbm.at[b], cost_v)

        for k in range(_NCHUNK):
            ds = pl.ds(k * 16, 16)
            v_v[ds] = jnp.zeros((16,), jnp.float32)
            p_v[ds] = jnp.zeros((16,), jnp.int32)
            way_v[ds] = jnp.zeros((16,), jnp.int32)
        for k in range(_NP // 16):
            u_v[pl.ds(k * 16, 16)] = jnp.zeros((16,), jnp.float32)

        def row_step(i, _):
            for k in range(_NCHUNK):
                ds = pl.ds(k * 16, 16)
                minv_v[ds] = jnp.full((16,), INF, jnp.float32)
                used_v[ds] = jnp.zeros((16,), jnp.int32)
            for k in range(_NP // 16):
                ds = pl.ds(k * 16, 16)
                io = lax.iota(jnp.int32, 16) + k * 16
                urow_v[ds] = jnp.where(io == i, 1, 0)

            scal_s[0] = jnp.int32(0)
            scal_s[1] = i

            def sbody(it, _):
                i0 = scal_s[1]

                @pl.when(i0 != 0)
                def _():
                    j0 = scal_s[0]
                    _sc_sstore(used_v, j0, 1, jnp.int32)
                    _sc_sstore(urow_v, i0, 1, jnp.int32)
                    ui0 = _sc_sload(u_v, i0)
                    delta = INF
                    j1 = jnp.int32(0)
                    for k in range(_NCHUNK):
                        ds = pl.ds(k * 16, 16)
                        cchunk = cost_v[i0 - 1, ds]
                        vcur = cchunk - ui0 - v_v[ds]
                        uch = used_v[ds]
                        mch = minv_v[ds]
                        mask = (uch == 0) & (vcur < mch)
                        mch = jnp.where(mask, vcur, mch)
                        minv_v[ds] = mch
                        way_v[ds] = jnp.where(mask, j0, way_v[ds])
                        cand = jnp.where(uch == 0, mch, INF)
                        cmin = jnp.min(cand)
                        cidx = jnp.min(plsc.all_reduce_ffs(cand == cmin))
                        better = cmin < delta
                        j1 = jnp.where(better, cidx + k * 16, j1)
                        delta = jnp.where(better, cmin, delta)
                    for k in range(_NP // 16):
                        ds = pl.ds(k * 16, 16)
                        u_v[ds] = u_v[ds] + jnp.where(
                            urow_v[ds] != 0, delta, jnp.float32(0.0))
                    for k in range(_NCHUNK):
                        ds = pl.ds(k * 16, 16)
                        um = used_v[ds] != 0
                        vv = v_v[ds]
                        mv = minv_v[ds]
                        v_v[ds] = jnp.where(um, vv - delta, vv)
                        minv_v[ds] = jnp.where(um, mv, mv - delta)
                    scal_s[0] = j1
                    scal_s[1] = _sc_sload(p_v, j1)
                return 0

            lax.fori_loop(0, i, sbody, 0)

            def bbody(it, _):
                j0c = scal_s[0]

                @pl.when(j0c != 0)
                def _():
                    j1w = _sc_sload(way_v, j0c)
                    pj = jnp.where(j1w == 0, i, _sc_sload(p_v, j1w))
                    _sc_sstore(p_v, j0c, pj, jnp.int32)
                    scal_s[0] = j1w
                return 0

            lax.fori_loop(0, i, bbody, 0)
            return 0

        lax.fori_loop(1, _T + 1, row_step, 0)

        for k in range(_NCHUNK):
            ds = pl.ds(k * 16, 16)
            pj = p_v[ds]
            idx = lax.iota(jnp.int32, 16) + k * 16
            mask = (pj > 0) & (idx >= 1) & (idx <= _Q)
            plsc.store_scatter(ans_v, [pj - 1], idx - 1, mask=mask)
        pltpu.sync_copy(ans_v, out_hbm.at[b])


def _sc_hungarian(cost):
    mesh = plsc.VectorSubcoreMesh(core_axis_name="c", subcore_axis_name="s")
    f = functools.partial(
        pl.kernel,
        mesh=mesh,
        compiler_params=pltpu.CompilerParams(needs_layout_passes=False),
        out_type=jax.ShapeDtypeStruct((_B, _NP), jnp.int32),
        scratch_types=[
            pltpu.VMEM((_NP, _MP), jnp.float32),
            pltpu.VMEM((_NP,), jnp.float32),
            pltpu.VMEM((_MP,), jnp.float32),
            pltpu.VMEM((_MP,), jnp.int32),
            pltpu.VMEM((_MP,), jnp.int32),
            pltpu.VMEM((_MP,), jnp.float32),
            pltpu.VMEM((_MP,), jnp.int32),
            pltpu.VMEM((_NP,), jnp.int32),
            pltpu.VMEM((_NP,), jnp.int32),
            pltpu.SMEM((2,), jnp.int32),
        ],
    )(_sc_hung_body)
    return f(cost)


def _loss_kernel(src_ref, nll_lab_ref, nll_eos_ref, bbt_ref, tbt_ref,
                 out_total, out_ce, out_bb, out_gi):
    colq = jax.lax.broadcasted_iota(jnp.int32, (_B, _Q), 1)
    tbt = tbt_ref[...]

    ce_q = jnp.zeros((_B, _Q), jnp.float32)
    matched = jnp.zeros((_B, _Q), jnp.bool_)
    tbq = [jnp.zeros((_B, _Q), jnp.float32) for _ in range(4)]
    for t in range(_T):
        sel = colq == src_ref[:, t:t + 1]
        matched = matched | sel
        ce_q = jnp.where(sel, nll_lab_ref[:, t, :], ce_q)
        for k in range(4):
            tbq[k] = jnp.where(sel, tbt[k][:, t:t + 1], tbq[k])

    nll_eos = nll_eos_ref[:, 0, :]
    ce_matched = jnp.sum(ce_q)
    eos_matched = jnp.sum(jnp.where(matched, nll_eos, 0.0))
    eos_all = jnp.sum(nll_eos)
    denom = _EOS * (_B * _Q - _B * _T) + 1.0 * (_B * _T)
    loss_ce = (_EOS * (eos_all - eos_matched) + ce_matched) / denom

    bb = [bbt_ref[k] for k in range(4)]
    l1 = jnp.zeros((_B, _Q), jnp.float32)
    for k in range(4):
        l1 = l1 + jnp.abs(bb[k] - tbq[k])
    loss_bbox = jnp.sum(jnp.where(matched, l1, 0.0)) / _B

    sx1 = bb[0] - 0.5 * bb[2]
    sy1 = bb[1] - 0.5 * bb[3]
    sx2 = bb[0] + 0.5 * bb[2]
    sy2 = bb[1] + 0.5 * bb[3]
    tx1 = tbq[0] - 0.5 * tbq[2]
    ty1 = tbq[1] - 0.5 * tbq[3]
    tx2 = tbq[0] + 0.5 * tbq[2]
    ty2 = tbq[1] + 0.5 * tbq[3]
    area1 = (sx2 - sx1) * (sy2 - sy1)
    area2 = (tx2 - tx1) * (ty2 - ty1)
    inter = (jnp.clip(jnp.minimum(sx2, tx2) - jnp.maximum(sx1, tx1), 0.0)
             * jnp.clip(jnp.minimum(sy2, ty2) - jnp.maximum(sy1, ty1), 0.0))
    union = area1 + area2 - inter
    iou = inter / union
    areai = (jnp.clip(jnp.maximum(sx2, tx2) - jnp.minimum(sx1, tx1), 0.0)
             * jnp.clip(jnp.maximum(sy2, ty2) - jnp.minimum(sy1, ty1), 0.0))
    giou = iou - (areai - union) / areai
    loss_giou = jnp.sum(jnp.where(matched, 1.0 - giou, 0.0)) / _B

    total = _W_CE * loss_ce + _W_BBOX * loss_bbox + _W_GIOU * loss_giou
    out_total[...] = total.reshape(1, 1)
    out_ce[...] = loss_ce.reshape(1, 1)
    out_bb[...] = loss_bbox.reshape(1, 1)
    out_gi[...] = loss_giou.reshape(1, 1)


def kernel(class_logits, bbox_pred, tgt_labels, tgt_boxes):
    lab3 = tgt_labels.astype(jnp.int32).reshape(_B, 1, _T)
    tbt = jnp.transpose(tgt_boxes, (0, 2, 1))

    cost, nll_lab, nll_eos = pl.pallas_call(
        _prep_kernel,
        grid=(_B // _IMG,),
        in_specs=[
            pl.BlockSpec((_IMG, _Q, _C1), lambda b: (b, 0, 0)),
            pl.BlockSpec((_IMG, _Q, 4), lambda b: (b, 0, 0)),
            pl.BlockSpec((_IMG, 1, _T), lambda b: (b, 0, 0)),
            pl.BlockSpec((_IMG, 4, _T), lambda b: (b, 0, 0)),
        ],
        out_specs=[
            pl.BlockSpec((_IMG, _NP, _MP), lambda b: (b, 0, 0)),
            pl.BlockSpec((_IMG, _T, _Q), lambda b: (b, 0, 0)),
            pl.BlockSpec((_IMG, 1, _Q), lambda b: (b, 0, 0)),
        ],
        out_shape=[
            jax.ShapeDtypeStruct((_B, _NP, _MP), jnp.float32),
            jax.ShapeDtypeStruct((_B, _T, _Q), jnp.float32),
            jax.ShapeDtypeStruct((_B, 1, _Q), jnp.float32),
        ],
        interpret=_INTERPRET,
    )(class_logits, bbox_pred, lab3, tbt)

    src = _sc_hungarian(cost)

    bbt = jnp.transpose(bbox_pred, (2, 0, 1))
    tbt2 = jnp.transpose(tgt_boxes, (2, 0, 1))

    outs = pl.pallas_call(
        _loss_kernel,
        out_shape=[jax.ShapeDtypeStruct((1, 1), jnp.float32)] * 4,
        interpret=_INTERPRET,
    )(src, nll_lab, nll_eos, bbt, tbt2)

    total, ce, bb, gi = [o[0, 0] for o in outs]
    return (total, ce, bb, gi)


# speedup vs baseline: 36.1958x; 1.2446x over previous
import functools

import jax
import jax.numpy as jnp
from jax import lax
from jax.experimental import pallas as pl
from jax.experimental.pallas import tpu as pltpu
from jax.experimental.pallas import tpu_sc as plsc

_NUM_CLASSES = 365
_C1 = _NUM_CLASSES + 1
_B, _Q, _T = 64, 300, 25
_EOS = 0.1
_W_CE, _W_BBOX, _W_GIOU = 1.0, 5.0, 2.0
_INF = 1e18
_MP = 304
_NP = 32
_NCHUNK = _MP // 16

_INTERPRET = False


_IMG = 4


def _prep_one(z, bbT, tb, lab, cost_ref, nll_lab_ref, nll_eos_ref, img):
    m = jnp.max(z, axis=1, keepdims=True)
    e = jnp.exp(z - m)
    s = jnp.sum(e, axis=1, keepdims=True)
    logs = jnp.log(s)

    cls_iota = jax.lax.broadcasted_iota(jnp.int32, (_C1, _T), 0)
    onehot = (cls_iota == lab).astype(jnp.bfloat16)
    h1 = z.astype(jnp.bfloat16)
    r1 = z - h1.astype(jnp.float32)
    h2 = r1.astype(jnp.bfloat16)
    h3 = (r1 - h2.astype(jnp.float32)).astype(jnp.bfloat16)
    lablogit = (jax.lax.dot(h1, onehot, preferred_element_type=jnp.float32)
                + jax.lax.dot(h2, onehot, preferred_element_type=jnp.float32)
                + jax.lax.dot(h3, onehot, preferred_element_type=jnp.float32))
    cost_classT = (-(jnp.exp(lablogit - m) / s)).T

    tx = tb[:, 0:1]
    ty = tb[:, 1:2]
    tw = tb[:, 2:3]
    th = tb[:, 3:4]
    bx = bbT[0:1, :]
    by = bbT[1:2, :]
    bw = bbT[2:3, :]
    bh = bbT[3:4, :]
    cost_bboxT = (jnp.abs(bx - tx) + jnp.abs(by - ty)
                  + jnp.abs(bw - tw) + jnp.abs(bh - th))

    sx1 = bx - 0.5 * bw
    sy1 = by - 0.5 * bh
    sx2 = bx + 0.5 * bw
    sy2 = by + 0.5 * bh
    tx1 = tx - 0.5 * tw
    ty1 = ty - 0.5 * th
    tx2 = tx + 0.5 * tw
    ty2 = ty + 0.5 * th
    area1 = (sx2 - sx1) * (sy2 - sy1)
    area2 = (tx2 - tx1) * (ty2 - ty1)
    ltx = jnp.maximum(sx1, tx1)
    lty = jnp.maximum(sy1, ty1)
    rbx = jnp.minimum(sx2, tx2)
    rby = jnp.minimum(sy2, ty2)
    inter = jnp.clip(rbx - ltx, 0.0) * jnp.clip(rby - lty, 0.0)
    union = area1 + area2 - inter
    iou = inter / union
    ltix = jnp.minimum(sx1, tx1)
    ltiy = jnp.minimum(sy1, ty1)
    rbix = jnp.maximum(sx2, tx2)
    rbiy = jnp.maximum(sy2, ty2)
    areai = jnp.clip(rbix - ltix, 0.0) * jnp.clip(rbiy - ltiy, 0.0)
    cost_giouT = -(iou - (areai - union) / areai)

    costT = (_W_BBOX * cost_bboxT + _W_CE * cost_classT
             + _W_GIOU * cost_giouT)

    cost_ref[img] = jnp.full((_NP, _MP), _INF, jnp.float32)
    cost_ref[img, 0:_T, 1:_Q + 1] = costT

    nll_lab_ref[img] = -(lablogit - m - logs).T
    zeos = z[:, _C1 - 1:_C1]
    nll_eos_ref[img] = -(zeos - m - logs).T


def _prep_kernel(logits_ref, bbt_ref, tb_ref, lab_ref,
                 cost_ref, nll_lab_ref, nll_eos_ref):
    for img in range(_IMG):
        _prep_one(logits_ref[img], bbt_ref[img], tb_ref[img], lab_ref[img],
                  cost_ref, nll_lab_ref, nll_eos_ref, img)


def _sc_sstore(ref, idx, val, dtype):
    lane0 = lax.iota(jnp.int32, 16) == 0
    iv = jnp.zeros((16,), jnp.int32) + idx
    xv = jnp.zeros((16,), dtype) + val
    plsc.store_scatter(ref, [iv], xv, mask=lane0)


def _sc_sload(ref, idx):
    iv = jnp.zeros((16,), jnp.int32) + idx
    return plsc.load_gather(ref, [iv])[0]


def _sc_hung_body(cost_hbm, out_hbm, cost_v, u_v, v_v, p_v, way_v,
                  minv_v, used_v, urow_v, ans_v, scal_s):
    INF = jnp.float32(_INF)
    wid = lax.axis_index("s") * 2 + lax.axis_index("c")

    for img in range(2):
        b = wid * 2 + img
        pltpu.sync_copy(cost_hbm.at[b], cost_v)

        for k in range(_NCHUNK):
            ds = pl.ds(k * 16, 16)
            v_v[ds] = jnp.zeros((16,), jnp.float32)
            p_v[ds] = jnp.zeros((16,), jnp.int32)
            way_v[ds] = jnp.zeros((16,), jnp.int32)
        for k in range(_NP // 16):
            u_v[pl.ds(k * 16, 16)] = jnp.zeros((16,), jnp.float32)

        def row_step(i, _):
            for k in range(_NCHUNK):
                ds = pl.ds(k * 16, 16)
                minv_v[ds] = jnp.full((16,), INF, jnp.float32)
                used_v[ds] = jnp.zeros((16,), jnp.int32)
            for k in range(_NP // 16):
                ds = pl.ds(k * 16, 16)
                io = lax.iota(jnp.int32, 16) + k * 16
                urow_v[ds] = jnp.where(io == i, 1, 0)

            scal_s[0] = jnp.int32(0)
            scal_s[1] = i

            def sbody(it, _):
                i0 = scal_s[1]

                @pl.when(i0 != 0)
                def _():
                    j0 = scal_s[0]
                    _sc_sstore(used_v, j0, 1, jnp.int32)
                    _sc_sstore(urow_v, i0, 1, jnp.int32)
                    ui0 = _sc_sload(u_v, i0)
                    delta = INF
                    j1 = jnp.int32(0)
                    for k in range(_NCHUNK):
                        ds = pl.ds(k * 16, 16)
                        cchunk = cost_v[i0 - 1, ds]
                        vcur = cchunk - ui0 - v_v[ds]
                        uch = used_v[ds]
                        mch = minv_v[ds]
                        mask = (uch == 0) & (vcur < mch)
                        mch = jnp.where(mask, vcur, mch)
                        minv_v[ds] = mch
                        way_v[ds] = jnp.where(mask, j0, way_v[ds])
                        cand = jnp.where(uch == 0, mch, INF)
                        cmin = jnp.min(cand)
                        cidx = jnp.min(plsc.all_reduce_ffs(cand == cmin))
                        better = cmin < delta
                        j1 = jnp.where(better, cidx + k * 16, j1)
                        delta = jnp.where(better, cmin, delta)
                    for k in range(_NP // 16):
                        ds = pl.ds(k * 16, 16)
                        u_v[ds] = u_v[ds] + jnp.where(
                            urow_v[ds] != 0, delta, jnp.float32(0.0))
                    for k in range(_NCHUNK):
                        ds = pl.ds(k * 16, 16)
                        um = used_v[ds] != 0
                        vv = v_v[ds]
                        mv = minv_v[ds]
                        v_v[ds] = jnp.where(um, vv - delta, vv)
                        minv_v[ds] = jnp.where(um, mv, mv - delta)
                    scal_s[0] = j1
                    scal_s[1] = _sc_sload(p_v, j1)
                return 0

            lax.fori_loop(0, i, sbody, 0)

            def bbody(it, _):
                j0c = scal_s[0]

                @pl.when(j0c != 0)
                def _():
                    j1w = _sc_sload(way_v, j0c)
                    pj = jnp.where(j1w == 0, i, _sc_sload(p_v, j1w))
                    _sc_sstore(p_v, j0c, pj, jnp.int32)
                    scal_s[0] = j1w
                return 0

            lax.fori_loop(0, i, bbody, 0)
            return 0

        lax.fori_loop(1, _T + 1, row_step, 0)

        for k in range(_NCHUNK):
            ds = pl.ds(k * 16, 16)
            pj = p_v[ds]
            idx = lax.iota(jnp.int32, 16) + k * 16
            mask = (pj > 0) & (idx >= 1) & (idx <= _Q)
            plsc.store_scatter(ans_v, [pj - 1], idx - 1, mask=mask)
        pltpu.sync_copy(ans_v, out_hbm.at[b])


def _sc_hungarian(cost):
    mesh = plsc.VectorSubcoreMesh(core_axis_name="c", subcore_axis_name="s")
    f = functools.partial(
        pl.kernel,
        mesh=mesh,
        compiler_params=pltpu.CompilerParams(needs_layout_passes=False),
        out_type=jax.ShapeDtypeStruct((_B, _NP), jnp.int32),
        scratch_types=[
            pltpu.VMEM((_NP, _MP), jnp.float32),
            pltpu.VMEM((_NP,), jnp.float32),
            pltpu.VMEM((_MP,), jnp.float32),
            pltpu.VMEM((_MP,), jnp.int32),
            pltpu.VMEM((_MP,), jnp.int32),
            pltpu.VMEM((_MP,), jnp.float32),
            pltpu.VMEM((_MP,), jnp.int32),
            pltpu.VMEM((_NP,), jnp.int32),
            pltpu.VMEM((_NP,), jnp.int32),
            pltpu.SMEM((2,), jnp.int32),
        ],
    )(_sc_hung_body)
    return f(cost)


def _loss_kernel(src_ref, nll_lab_ref, nll_eos_ref, bbt_ref, tbt_ref,
                 out_total, out_ce, out_bb, out_gi):
    colq = jax.lax.broadcasted_iota(jnp.int32, (_B, _Q), 1)
    tbt = tbt_ref[...]

    ce_q = jnp.zeros((_B, _Q), jnp.float32)
    matched = jnp.zeros((_B, _Q), jnp.bool_)
    tbq = [jnp.zeros((_B, _Q), jnp.float32) for _ in range(4)]
    for t in range(_T):
        sel = colq == src_ref[:, t:t + 1]
        matched = matched | sel
        ce_q = jnp.where(sel, nll_lab_ref[:, t, :], ce_q)
        for k in range(4):
            tbq[k] = jnp.where(sel, tbt[k][:, t:t + 1], tbq[k])

    nll_eos = nll_eos_ref[:, 0, :]
    ce_matched = jnp.sum(ce_q)
    eos_matched = jnp.sum(jnp.where(matched, nll_eos, 0.0))
    eos_all = jnp.sum(nll_eos)
    denom = _EOS * (_B * _Q - _B * _T) + 1.0 * (_B * _T)
    loss_ce = (_EOS * (eos_all - eos_matched) + ce_matched) / denom

    bb = [bbt_ref[k] for k in range(4)]
    l1 = jnp.zeros((_B, _Q), jnp.float32)
    for k in range(4):
        l1 = l1 + jnp.abs(bb[k] - tbq[k])
    loss_bbox = jnp.sum(jnp.where(matched, l1, 0.0)) / _B

    sx1 = bb[0] - 0.5 * bb[2]
    sy1 = bb[1] - 0.5 * bb[3]
    sx2 = bb[0] + 0.5 * bb[2]
    sy2 = bb[1] + 0.5 * bb[3]
    tx1 = tbq[0] - 0.5 * tbq[2]
    ty1 = tbq[1] - 0.5 * tbq[3]
    tx2 = tbq[0] + 0.5 * tbq[2]
    ty2 = tbq[1] + 0.5 * tbq[3]
    area1 = (sx2 - sx1) * (sy2 - sy1)
    area2 = (tx2 - tx1) * (ty2 - ty1)
    inter = (jnp.clip(jnp.minimum(sx2, tx2) - jnp.maximum(sx1, tx1), 0.0)
             * jnp.clip(jnp.minimum(sy2, ty2) - jnp.maximum(sy1, ty1), 0.0))
    union = area1 + area2 - inter
    iou = inter / union
    areai = (jnp.clip(jnp.maximum(sx2, tx2) - jnp.minimum(sx1, tx1), 0.0)
             * jnp.clip(jnp.maximum(sy2, ty2) - jnp.minimum(sy1, ty1), 0.0))
    giou = iou - (areai - union) / areai
    loss_giou = jnp.sum(jnp.where(matched, 1.0 - giou, 0.0)) / _B

    total = _W_CE * loss_ce + _W_BBOX * loss_bbox + _W_GIOU * loss_giou
    out_total[...] = total.reshape(1, 1)
    out_ce[...] = loss_ce.reshape(1, 1)
    out_bb[...] = loss_bbox.reshape(1, 1)
    out_gi[...] = loss_giou.reshape(1, 1)


def kernel(class_logits, bbox_pred, tgt_labels, tgt_boxes):
    lab3 = tgt_labels.astype(jnp.int32).reshape(_B, 1, _T)
    bbt4 = jnp.transpose(bbox_pred, (0, 2, 1))

    cost, nll_lab, nll_eos = pl.pallas_call(
        _prep_kernel,
        grid=(_B // _IMG,),
        in_specs=[
            pl.BlockSpec((_IMG, _Q, _C1), lambda b: (b, 0, 0)),
            pl.BlockSpec((_IMG, 4, _Q), lambda b: (b, 0, 0)),
            pl.BlockSpec((_IMG, _T, 4), lambda b: (b, 0, 0)),
            pl.BlockSpec((_IMG, 1, _T), lambda b: (b, 0, 0)),
        ],
        out_specs=[
            pl.BlockSpec((_IMG, _NP, _MP), lambda b: (b, 0, 0)),
            pl.BlockSpec((_IMG, _T, _Q), lambda b: (b, 0, 0)),
            pl.BlockSpec((_IMG, 1, _Q), lambda b: (b, 0, 0)),
        ],
        out_shape=[
            jax.ShapeDtypeStruct((_B, _NP, _MP), jnp.float32),
            jax.ShapeDtypeStruct((_B, _T, _Q), jnp.float32),
            jax.ShapeDtypeStruct((_B, 1, _Q), jnp.float32),
        ],
        interpret=_INTERPRET,
    )(class_logits, bbt4, tgt_boxes, lab3)

    src = _sc_hungarian(cost)

    bbt = jnp.transpose(bbox_pred, (2, 0, 1))
    tbt2 = jnp.transpose(tgt_boxes, (2, 0, 1))

    outs = pl.pallas_call(
        _loss_kernel,
        out_shape=[jax.ShapeDtypeStruct((1, 1), jnp.float32)] * 4,
        interpret=_INTERPRET,
    )(src, nll_lab, nll_eos, bbt, tbt2)

    total, ce, bb, gi = [o[0, 0] for o in outs]
    return (total, ce, bb, gi)


# speedup vs baseline: 41.1235x; 1.1361x over previous
import functools

import jax
import jax.numpy as jnp
from jax import lax
from jax.experimental import pallas as pl
from jax.experimental.pallas import tpu as pltpu
from jax.experimental.pallas import tpu_sc as plsc

_NUM_CLASSES = 365
_C1 = _NUM_CLASSES + 1
_B, _Q, _T = 64, 300, 25
_EOS = 0.1
_W_CE, _W_BBOX, _W_GIOU = 1.0, 5.0, 2.0
_INF = 1e18
_MP = 304
_NP = 32
_NCHUNK = _MP // 16

_INTERPRET = False


_IMG = 4


def _prep_one(z, bbT, tb, lab, cost_ref, nll_lab_ref, nll_eos_ref, img):
    m = jnp.max(z, axis=1, keepdims=True)
    e = jnp.exp(z - m)
    s = jnp.sum(e, axis=1, keepdims=True)
    logs = jnp.log(s)

    cls_iota = jax.lax.broadcasted_iota(jnp.int32, (_C1, _T), 0)
    onehot = (cls_iota == lab).astype(jnp.bfloat16)
    h1 = z.astype(jnp.bfloat16)
    r1 = z - h1.astype(jnp.float32)
    h2 = r1.astype(jnp.bfloat16)
    h3 = (r1 - h2.astype(jnp.float32)).astype(jnp.bfloat16)
    lablogit = (jax.lax.dot(h1, onehot, preferred_element_type=jnp.float32)
                + jax.lax.dot(h2, onehot, preferred_element_type=jnp.float32)
                + jax.lax.dot(h3, onehot, preferred_element_type=jnp.float32))
    cost_classT = (-(jnp.exp(lablogit - m) / s)).T

    tx = tb[:, 0:1]
    ty = tb[:, 1:2]
    tw = tb[:, 2:3]
    th = tb[:, 3:4]
    bx = bbT[0:1, :]
    by = bbT[1:2, :]
    bw = bbT[2:3, :]
    bh = bbT[3:4, :]
    cost_bboxT = (jnp.abs(bx - tx) + jnp.abs(by - ty)
                  + jnp.abs(bw - tw) + jnp.abs(bh - th))

    sx1 = bx - 0.5 * bw
    sy1 = by - 0.5 * bh
    sx2 = bx + 0.5 * bw
    sy2 = by + 0.5 * bh
    tx1 = tx - 0.5 * tw
    ty1 = ty - 0.5 * th
    tx2 = tx + 0.5 * tw
    ty2 = ty + 0.5 * th
    area1 = (sx2 - sx1) * (sy2 - sy1)
    area2 = (tx2 - tx1) * (ty2 - ty1)
    ltx = jnp.maximum(sx1, tx1)
    lty = jnp.maximum(sy1, ty1)
    rbx = jnp.minimum(sx2, tx2)
    rby = jnp.minimum(sy2, ty2)
    inter = jnp.clip(rbx - ltx, 0.0) * jnp.clip(rby - lty, 0.0)
    union = area1 + area2 - inter
    iou = inter / union
    ltix = jnp.minimum(sx1, tx1)
    ltiy = jnp.minimum(sy1, ty1)
    rbix = jnp.maximum(sx2, tx2)
    rbiy = jnp.maximum(sy2, ty2)
    areai = jnp.clip(rbix - ltix, 0.0) * jnp.clip(rbiy - ltiy, 0.0)
    cost_giouT = -(iou - (areai - union) / areai)

    costT = (_W_BBOX * cost_bboxT + _W_CE * cost_classT
             + _W_GIOU * cost_giouT)

    cost_ref[img] = jnp.full((_NP, _MP), _INF, jnp.float32)
    cost_ref[img, 0:_T, 1:_Q + 1] = costT

    nll_lab_ref[img] = -(lablogit - m - logs).T
    zeos = z[:, _C1 - 1:_C1]
    nll_eos_ref[img] = -(zeos - m - logs).T


def _prep_kernel(logits_ref, bbt_ref, tb_ref, lab_ref,
                 cost_ref, nll_lab_ref, nll_eos_ref):
    for img in range(_IMG):
        _prep_one(logits_ref[img], bbt_ref[img], tb_ref[img], lab_ref[img],
                  cost_ref, nll_lab_ref, nll_eos_ref, img)


def _sc_sstore(ref, idx, val, dtype):
    lane0 = lax.iota(jnp.int32, 16) == 0
    iv = jnp.zeros((16,), jnp.int32) + idx
    xv = jnp.zeros((16,), dtype) + val
    plsc.store_scatter(ref, [iv], xv, mask=lane0)


def _sc_sload(ref, idx):
    iv = jnp.zeros((16,), jnp.int32) + idx
    return plsc.load_gather(ref, [iv])[0]


def _sc_hung_body(ipw, cost_hbm, out_hbm, cost_v, u_v, v_v, p_v, way_v,
                  minv_v, used_v, urow_v, ans_v, scal_s):
    INF = jnp.float32(_INF)
    wid = lax.axis_index("s") * 2 + lax.axis_index("c")

    for img in range(ipw):
        b = wid * ipw + img
        pltpu.sync_copy(cost_hbm.at[b], cost_v)

        for k in range(_NCHUNK):
            ds = pl.ds(k * 16, 16)
            v_v[ds] = jnp.zeros((16,), jnp.float32)
            p_v[ds] = jnp.zeros((16,), jnp.int32)
            way_v[ds] = jnp.zeros((16,), jnp.int32)
        for k in range(_NP // 16):
            u_v[pl.ds(k * 16, 16)] = jnp.zeros((16,), jnp.float32)

        def row_step(i, _):
            for k in range(_NCHUNK):
                ds = pl.ds(k * 16, 16)
                minv_v[ds] = jnp.full((16,), INF, jnp.float32)
                used_v[ds] = jnp.zeros((16,), jnp.int32)
            for k in range(_NP // 16):
                ds = pl.ds(k * 16, 16)
                io = lax.iota(jnp.int32, 16) + k * 16
                urow_v[ds] = jnp.where(io == i, 1, 0)

            scal_s[0] = jnp.int32(0)
            scal_s[1] = i

            def sbody(it, _):
                i0 = scal_s[1]

                @pl.when(i0 != 0)
                def _():
                    j0 = scal_s[0]
                    _sc_sstore(used_v, j0, 1, jnp.int32)
                    _sc_sstore(urow_v, i0, 1, jnp.int32)
                    ui0 = _sc_sload(u_v, i0)
                    delta = INF
                    j1 = jnp.int32(0)
                    for k in range(_NCHUNK):
                        ds = pl.ds(k * 16, 16)
                        cchunk = cost_v[i0 - 1, ds]
                        vcur = cchunk - ui0 - v_v[ds]
                        uch = used_v[ds]
                        mch = minv_v[ds]
                        mask = (uch == 0) & (vcur < mch)
                        mch = jnp.where(mask, vcur, mch)
                        minv_v[ds] = mch
                        way_v[ds] = jnp.where(mask, j0, way_v[ds])
                        cand = jnp.where(uch == 0, mch, INF)
                        cmin = jnp.min(cand)
                        cidx = jnp.min(plsc.all_reduce_ffs(cand == cmin))
                        better = cmin < delta
                        j1 = jnp.where(better, cidx + k * 16, j1)
                        delta = jnp.where(better, cmin, delta)
                    for k in range(_NP // 16):
                        ds = pl.ds(k * 16, 16)
                        u_v[ds] = u_v[ds] + jnp.where(
                            urow_v[ds] != 0, delta, jnp.float32(0.0))
                    for k in range(_NCHUNK):
                        ds = pl.ds(k * 16, 16)
                        um = used_v[ds] != 0
                        vv = v_v[ds]
                        mv = minv_v[ds]
                        v_v[ds] = jnp.where(um, vv - delta, vv)
                        minv_v[ds] = jnp.where(um, mv, mv - delta)
                    scal_s[0] = j1
                    scal_s[1] = _sc_sload(p_v, j1)
                return 0

            lax.fori_loop(0, i, sbody, 0)

            def bbody(it, _):
                j0c = scal_s[0]

                @pl.when(j0c != 0)
                def _():
                    j1w = _sc_sload(way_v, j0c)
                    pj = jnp.where(j1w == 0, i, _sc_sload(p_v, j1w))
                    _sc_sstore(p_v, j0c, pj, jnp.int32)
                    scal_s[0] = j1w
                return 0

            lax.fori_loop(0, i, bbody, 0)
            return 0

        lax.fori_loop(1, _T + 1, row_step, 0)

        for k in range(_NCHUNK):
            ds = pl.ds(k * 16, 16)
            pj = p_v[ds]
            idx = lax.iota(jnp.int32, 16) + k * 16
            mask = (pj > 0) & (idx >= 1) & (idx <= _Q)
            plsc.store_scatter(ans_v, [pj - 1], idx - 1, mask=mask)
        pltpu.sync_copy(ans_v, out_hbm.at[b])


def _sc_hungarian(cost):
    nb = cost.shape[0]
    mesh = plsc.VectorSubcoreMesh(core_axis_name="c", subcore_axis_name="s")
    f = functools.partial(
        pl.kernel,
        mesh=mesh,
        compiler_params=pltpu.CompilerParams(needs_layout_passes=False),
        out_type=jax.ShapeDtypeStruct((nb, _NP), jnp.int32),
        scratch_types=[
            pltpu.VMEM((_NP, _MP), jnp.float32),
            pltpu.VMEM((_NP,), jnp.float32),
            pltpu.VMEM((_MP,), jnp.float32),
            pltpu.VMEM((_MP,), jnp.int32),
            pltpu.VMEM((_MP,), jnp.int32),
            pltpu.VMEM((_MP,), jnp.float32),
            pltpu.VMEM((_MP,), jnp.int32),
            pltpu.VMEM((_NP,), jnp.int32),
            pltpu.VMEM((_NP,), jnp.int32),
            pltpu.SMEM((2,), jnp.int32),
        ],
    )(functools.partial(_sc_hung_body, nb // 32))
    return f(cost)


_BH = _B // 2


def _loss_half(src_ref, nll_lab_ref, nll_eos_ref, bbt_ref, tbt_ref, h):
    base = h * _BH
    colq = jax.lax.broadcasted_iota(jnp.int32, (_BH, _Q), 1)
    tbt = tbt_ref[:, base:base + _BH, :]

    ce_q = jnp.zeros((_BH, _Q), jnp.float32)
    matched = jnp.zeros((_BH, _Q), jnp.bool_)
    tbq = [jnp.zeros((_BH, _Q), jnp.float32) for _ in range(4)]
    for t in range(_T):
        sel = colq == src_ref[:, t:t + 1]
        matched = matched | sel
        ce_q = jnp.where(sel, nll_lab_ref[:, t, :], ce_q)
        for k in range(4):
            tbq[k] = jnp.where(sel, tbt[k][:, t:t + 1], tbq[k])

    nll_eos = nll_eos_ref[:, 0, :]
    ce_matched = jnp.sum(ce_q)
    eos_matched = jnp.sum(jnp.where(matched, nll_eos, 0.0))
    eos_all = jnp.sum(nll_eos)

    bb = [bbt_ref[k, base:base + _BH, :] for k in range(4)]
    l1 = jnp.zeros((_BH, _Q), jnp.float32)
    for k in range(4):
        l1 = l1 + jnp.abs(bb[k] - tbq[k])
    l1_sum = jnp.sum(jnp.where(matched, l1, 0.0))

    sx1 = bb[0] - 0.5 * bb[2]
    sy1 = bb[1] - 0.5 * bb[3]
    sx2 = bb[0] + 0.5 * bb[2]
    sy2 = bb[1] + 0.5 * bb[3]
    tx1 = tbq[0] - 0.5 * tbq[2]
    ty1 = tbq[1] - 0.5 * tbq[3]
    tx2 = tbq[0] + 0.5 * tbq[2]
    ty2 = tbq[1] + 0.5 * tbq[3]
    area1 = (sx2 - sx1) * (sy2 - sy1)
    area2 = (tx2 - tx1) * (ty2 - ty1)
    inter = (jnp.clip(jnp.minimum(sx2, tx2) - jnp.maximum(sx1, tx1), 0.0)
             * jnp.clip(jnp.minimum(sy2, ty2) - jnp.maximum(sy1, ty1), 0.0))
    union = area1 + area2 - inter
    iou = inter / union
    areai = (jnp.clip(jnp.maximum(sx2, tx2) - jnp.minimum(sx1, tx1), 0.0)
             * jnp.clip(jnp.maximum(sy2, ty2) - jnp.minimum(sy1, ty1), 0.0))
    giou = iou - (areai - union) / areai
    giou_sum = jnp.sum(jnp.where(matched, 1.0 - giou, 0.0))
    return ce_matched, eos_matched, eos_all, l1_sum, giou_sum


def _loss_kernel(src_a, src_b, nll_a, nll_b, eos_a, eos_b, bbt_ref, tbt_ref,
                 out_total, out_ce, out_bb, out_gi):
    pa = _loss_half(src_a, nll_a, eos_a, bbt_ref, tbt_ref, 0)
    pb = _loss_half(src_b, nll_b, eos_b, bbt_ref, tbt_ref, 1)
    ce_matched = pa[0] + pb[0]
    eos_matched = pa[1] + pb[1]
    eos_all = pa[2] + pb[2]
    denom = _EOS * (_B * _Q - _B * _T) + 1.0 * (_B * _T)
    loss_ce = (_EOS * (eos_all - eos_matched) + ce_matched) / denom
    loss_bbox = (pa[3] + pb[3]) / _B
    loss_giou = (pa[4] + pb[4]) / _B

    total = _W_CE * loss_ce + _W_BBOX * loss_bbox + _W_GIOU * loss_giou
    out_total[...] = total.reshape(1, 1)
    out_ce[...] = loss_ce.reshape(1, 1)
    out_bb[...] = loss_bbox.reshape(1, 1)
    out_gi[...] = loss_giou.reshape(1, 1)


def kernel(class_logits, bbox_pred, tgt_labels, tgt_boxes):
    lab3 = tgt_labels.astype(jnp.int32).reshape(_B, 1, _T)
    bbt4 = jnp.transpose(bbox_pred, (0, 2, 1))

    def prep_half(off):
        return pl.pallas_call(
            _prep_kernel,
            grid=(_BH // _IMG,),
            in_specs=[
                pl.BlockSpec((_IMG, _Q, _C1), lambda b: (b + off, 0, 0)),
                pl.BlockSpec((_IMG, 4, _Q), lambda b: (b + off, 0, 0)),
                pl.BlockSpec((_IMG, _T, 4), lambda b: (b + off, 0, 0)),
                pl.BlockSpec((_IMG, 1, _T), lambda b: (b + off, 0, 0)),
            ],
            out_specs=[
                pl.BlockSpec((_IMG, _NP, _MP), lambda b: (b, 0, 0)),
                pl.BlockSpec((_IMG, _T, _Q), lambda b: (b, 0, 0)),
                pl.BlockSpec((_IMG, 1, _Q), lambda b: (b, 0, 0)),
            ],
            out_shape=[
                jax.ShapeDtypeStruct((_BH, _NP, _MP), jnp.float32),
                jax.ShapeDtypeStruct((_BH, _T, _Q), jnp.float32),
                jax.ShapeDtypeStruct((_BH, 1, _Q), jnp.float32),
            ],
            interpret=_INTERPRET,
        )(class_logits, bbt4, tgt_boxes, lab3)

    cost_a, nll_a, eos_a = prep_half(0)
    src_a = _sc_hungarian(cost_a)
    cost_b, nll_b, eos_b = prep_half(_BH // _IMG)
    src_b = _sc_hungarian(cost_b)

    bbt = jnp.transpose(bbox_pred, (2, 0, 1))
    tbt2 = jnp.transpose(tgt_boxes, (2, 0, 1))

    outs = pl.pallas_call(
        _loss_kernel,
        out_shape=[jax.ShapeDtypeStruct((1, 1), jnp.float32)] * 4,
        interpret=_INTERPRET,
    )(src_a, src_b, nll_a, nll_b, eos_a, eos_b, bbt, tbt2)

    total, ce, bb, gi = [o[0, 0] for o in outs]
    return (total, ce, bb, gi)
